# baseline (speedup 1.0000x reference)
import jax
import jax.numpy as jnp
from jax.experimental import pallas as pl

NUM_CATEGORIES = 1000
C_SPLIT = 5
C_BLOCK = NUM_CATEGORIES // C_SPLIT


def _onehot_body(inp_ref, out_ref):
    f = pl.program_id(0)
    c = pl.program_id(1)
    v = inp_ref[pl.ds(f, 1), :]
    iota = jax.lax.broadcasted_iota(
        jnp.int32, (1, C_BLOCK, v.shape[1]), 1
    ) + c * C_BLOCK
    out_ref[...] = (iota == v[:, None, :]).astype(jnp.float32)


def kernel(inputs):
    batch, nfeat = inputs.shape
    vt = inputs.astype(jnp.int32).T
    out_t = pl.pallas_call(
        _onehot_body,
        grid=(nfeat, C_SPLIT),
        in_specs=[pl.BlockSpec((nfeat, batch), lambda f, c: (0, 0))],
        out_specs=pl.BlockSpec((1, C_BLOCK, batch), lambda f, c: (f, c, 0)),
        out_shape=jax.ShapeDtypeStruct((nfeat, NUM_CATEGORIES, batch), jnp.float32),
    )(vt)
    return jnp.transpose(out_t, (2, 0, 1))


# speedup vs baseline: 1.9294x; 1.9294x over previous
import jax
import jax.numpy as jnp
from jax.experimental import pallas as pl

NUM_CATEGORIES = 1000


def _onehot_body(inp_ref, out_ref):
    f = pl.program_id(0)
    v = inp_ref[pl.ds(f, 1), :]
    iota = jax.lax.broadcasted_iota(
        jnp.int32, (1, NUM_CATEGORIES, v.shape[1]), 1
    )
    out_ref[...] = (iota == v[:, None, :]).astype(jnp.float32)


def kernel(inputs):
    batch, nfeat = inputs.shape
    vt = inputs.astype(jnp.int32).T
    out_t = pl.pallas_call(
        _onehot_body,
        grid=(nfeat,),
        in_specs=[pl.BlockSpec((nfeat, batch), lambda f: (0, 0))],
        out_specs=pl.BlockSpec((1, NUM_CATEGORIES, batch), lambda f: (f, 0, 0)),
        out_shape=jax.ShapeDtypeStruct((nfeat, NUM_CATEGORIES, batch), jnp.float32),
    )(vt)
    return jnp.transpose(out_t, (2, 0, 1))
